# baseline (speedup 1.0000x reference)
import jax
import jax.numpy as jnp
from jax import lax
from jax.experimental import pallas as pl
from jax.experimental.pallas import tpu as pltpu
from jax.experimental.pallas import tpu_sc as plsc

N_NODES = 10000
N_EDGES = 320000
D = 128

NC = 2
NS = 16
LANES = 16
K = 128
E_PER_CORE = N_EDGES // NC
E_PER_TILE = E_PER_CORE // NS
N_CHUNKS = -(-E_PER_TILE // K)
E_PT_PAD = N_CHUNKS * K
N_PAD = 10240
ROWS_PER_TILE = N_PAD // NS
CW = 16


def _sc_aggregate(x, src, dst):
    mesh = plsc.VectorSubcoreMesh(core_axis_name="c", subcore_axis_name="s")

    def body(x_h, src_h, dst_h, part_x_h,
             acc_x, idx_s0, idx_d0, rows0, idx_s1, idx_d1, rows1,
             sem0, sem1):
        c = lax.axis_index("c")
        s = lax.axis_index("s")
        r0 = s * ROWS_PER_TILE
        e0 = (c * NS + s) * E_PT_PAD
        out0 = c * N_PAD + r0

        zv = jnp.zeros((LANES,), jnp.float32)

        def zrow(i, carry):
            for l in range(D // LANES):
                rows0[i, pl.ds(l * LANES, LANES)] = zv
            return carry

        lax.fori_loop(0, K, zrow, 0)

        for q in range(ROWS_PER_TILE // K):
            pltpu.sync_copy(rows0, acc_x.at[pl.ds(r0 + q * K, K)])
        plsc.subcore_barrier()

        def load(j, idx_s, idx_d, rows, sem):
            base = e0 + j * K
            pltpu.sync_copy(src_h.at[pl.ds(base, K)], idx_s)
            pltpu.sync_copy(dst_h.at[pl.ds(base, K)], idx_d)
            pltpu.async_copy(x_h.at[idx_s], rows, sem)

        def drain_scatter(idx_s, idx_d, rows, sem):
            pltpu.make_async_copy(x_h.at[idx_s], rows, sem).wait()
            pltpu.sync_copy(rows, acc_x.at[idx_d], add=True)

        load(0, idx_s0, idx_d0, rows0, sem0)

        def body2(t, carry):
            load(2 * t + 1, idx_s1, idx_d1, rows1, sem1)
            drain_scatter(idx_s0, idx_d0, rows0, sem0)

            @pl.when(2 * t + 2 < N_CHUNKS)
            def _():
                load(2 * t + 2, idx_s0, idx_d0, rows0, sem0)

            drain_scatter(idx_s1, idx_d1, rows1, sem1)
            return carry

        lax.fori_loop(0, N_CHUNKS // 2, body2, 0)
        drain_scatter(idx_s0, idx_d0, rows0, sem0)
        plsc.subcore_barrier()

        pltpu.sync_copy(acc_x.at[pl.ds(r0, ROWS_PER_TILE)],
                        part_x_h.at[pl.ds(out0, ROWS_PER_TILE)])


    call = pl.kernel(
        body,
        out_type=jax.ShapeDtypeStruct((NC * N_PAD, D), jnp.float32),
        mesh=mesh,
        scratch_types=[
            pltpu.VMEM_SHARED((N_PAD, D), jnp.float32),
            pltpu.VMEM((K,), jnp.int32),
            pltpu.VMEM((K,), jnp.int32),
            pltpu.VMEM((K, D), jnp.float32),
            pltpu.VMEM((K,), jnp.int32),
            pltpu.VMEM((K,), jnp.int32),
            pltpu.VMEM((K, D), jnp.float32),
            pltpu.SemaphoreType.DMA,
            pltpu.SemaphoreType.DMA,
        ],
    )
    return call(x, src, dst)


def _sc_count(dst):
    mesh = plsc.VectorSubcoreMesh(core_axis_name="c", subcore_axis_name="s")

    def body(dst_h, part_c_h, acc_c, idx_d0, idx_d1, rows, ones_v):
        c = lax.axis_index("c")
        s = lax.axis_index("s")
        r0 = s * ROWS_PER_TILE
        e0 = (c * NS + s) * E_PT_PAD
        out0 = c * N_PAD + r0

        zv = jnp.zeros((LANES,), jnp.float32)
        ov = jnp.ones((LANES,), jnp.float32)

        def zrow(i, carry):
            for l in range(D // LANES):
                rows[i, pl.ds(l * LANES, LANES)] = zv
                ones_v[i, pl.ds(l * LANES, LANES)] = ov
            return carry

        lax.fori_loop(0, K, zrow, 0)

        for q in range(ROWS_PER_TILE // K):
            pltpu.sync_copy(rows, acc_c.at[pl.ds(r0 + q * K, K)])
        plsc.subcore_barrier()

        pltpu.sync_copy(dst_h.at[pl.ds(e0, K)], idx_d0)

        def body2(t, carry):
            pltpu.sync_copy(dst_h.at[pl.ds(e0 + (2 * t + 1) * K, K)], idx_d1)
            pltpu.sync_copy(ones_v, acc_c.at[idx_d0], add=True)

            @pl.when(2 * t + 2 < N_CHUNKS)
            def _():
                pltpu.sync_copy(dst_h.at[pl.ds(e0 + (2 * t + 2) * K, K)],
                                idx_d0)

            pltpu.sync_copy(ones_v, acc_c.at[idx_d1], add=True)
            return carry

        lax.fori_loop(0, N_CHUNKS // 2, body2, 0)
        pltpu.sync_copy(ones_v, acc_c.at[idx_d0], add=True)
        plsc.subcore_barrier()

        pltpu.sync_copy(acc_c.at[pl.ds(r0, ROWS_PER_TILE)],
                        part_c_h.at[pl.ds(out0, ROWS_PER_TILE)])

    call = pl.kernel(
        body,
        out_type=jax.ShapeDtypeStruct((NC * N_PAD, D), jnp.float32),
        mesh=mesh,
        scratch_types=[
            pltpu.VMEM_SHARED((N_PAD, D), jnp.float32),
            pltpu.VMEM((K,), jnp.int32),
            pltpu.VMEM((K,), jnp.int32),
            pltpu.VMEM((K, D), jnp.float32),
            pltpu.VMEM((K, D), jnp.float32),
        ],
    )
    return call(dst)


def _tc_tail(px0, px1, pc0, pc1, x, WlT, bl, WrT, WlnT):
    def body(px0_ref, px1_ref, pc0_ref, pc1_ref, x_ref,
             wl_ref, bl_ref, wr_ref, wln_ref, o_ref):
        agg = px0_ref[...] + px1_ref[...]
        cnt = pc0_ref[...] + pc1_ref[...]
        mean = agg / jnp.maximum(cnt, 1.0)
        xb = x_ref[...]
        h = jnp.dot(mean, wl_ref[...], preferred_element_type=jnp.float32)
        h = h + bl_ref[...] + jnp.dot(xb, wr_ref[...],
                                      preferred_element_type=jnp.float32)
        h = jnp.maximum(h, 0.0)
        o = jnp.dot(h, wln_ref[...], preferred_element_type=jnp.float32) + xb
        o_ref[...] = jnp.maximum(o, 0.0)

    grid = (N_NODES // TC_BLK,)
    return pl.pallas_call(
        body,
        grid=grid,
        in_specs=[
            pl.BlockSpec((TC_BLK, D), lambda i: (i, 0)),
            pl.BlockSpec((TC_BLK, D), lambda i: (i, 0)),
            pl.BlockSpec((TC_BLK, D), lambda i: (i, 0)),
            pl.BlockSpec((TC_BLK, D), lambda i: (i, 0)),
            pl.BlockSpec((TC_BLK, D), lambda i: (i, 0)),
            pl.BlockSpec((D, D), lambda i: (0, 0)),
            pl.BlockSpec((1, D), lambda i: (0, 0)),
            pl.BlockSpec((D, D), lambda i: (0, 0)),
            pl.BlockSpec((D, D), lambda i: (0, 0)),
        ],
        out_specs=pl.BlockSpec((TC_BLK, D), lambda i: (i, 0)),
        out_shape=jax.ShapeDtypeStruct((N_NODES, D), jnp.float32),
    )(px0, px1, pc0, pc1, x, WlT, bl, WrT, WlnT)


TC_BLK = 1000


def kernel(x, edge_index, W_l, b_l, W_r, W_ln):
    src = edge_index[0].astype(jnp.int32)
    dst = edge_index[1].astype(jnp.int32)
    pad = E_PT_PAD - E_PER_TILE
    srcp = jnp.pad(src.reshape(NC * NS, E_PER_TILE),
                   ((0, 0), (0, pad))).reshape(-1)
    dstp = jnp.pad(dst.reshape(NC * NS, E_PER_TILE),
                   ((0, 0), (0, pad)),
                   constant_values=N_PAD - 1).reshape(-1)
    part_x = _sc_aggregate(x, srcp, dstp)
    part_c = _sc_count(dstp)
    px0, px1 = part_x[:N_PAD], part_x[N_PAD:]
    pc0, pc1 = part_c[:N_PAD], part_c[N_PAD:]
    return _tc_tail(px0, px1, pc0, pc1, x,
                    W_l.T, b_l.reshape(1, D), W_r.T, W_ln.T)


# speedup vs baseline: 1.2622x; 1.2622x over previous
import jax
import jax.numpy as jnp
from jax import lax
from jax.experimental import pallas as pl
from jax.experimental.pallas import tpu as pltpu
from jax.experimental.pallas import tpu_sc as plsc

N_NODES = 10000
N_EDGES = 320000
D = 128

NC = 2
NS = 16
LANES = 16
K = 80
E_PER_CORE = N_EDGES // NC
E_PER_TILE = E_PER_CORE // NS
N_CHUNKS = E_PER_TILE // K
E_PT_PAD = E_PER_TILE
NBUF = 2
N_BLK = 5
CPB = N_CHUNKS // N_BLK
N_PAD = 10240
ROWS_PER_TILE = N_PAD // NS
CW = 16


def _sc_aggregate(x, src4, dst4):
    mesh = plsc.VectorSubcoreMesh(core_axis_name="c", subcore_axis_name="s")

    def body(x_h, src_h, dst_h, part_x_h, acc_x,
             ibs, ibd, rows0, rows1, id0, id1, gsem0, gsem1, ssem0, ssem1):
        rows = (rows0, rows1)
        idxd = (id0, id1)
        gsem = (gsem0, gsem1)
        ssem = (ssem0, ssem1)

        c = lax.axis_index("c")
        s = lax.axis_index("s")
        wid = c * NS + s
        r0 = s * ROWS_PER_TILE
        out0 = c * N_PAD + r0

        zv = jnp.zeros((LANES,), jnp.float32)

        def zrow(i, carry):
            for l in range(D // LANES):
                rows0[i, pl.ds(l * LANES, LANES)] = zv
            return carry

        lax.fori_loop(0, K, zrow, 0)

        for q in range(ROWS_PER_TILE // K):
            pltpu.sync_copy(rows0, acc_x.at[pl.ds(r0 + q * K, K)])
        plsc.subcore_barrier()

        def drain(p):
            pltpu.make_async_copy(rows[p], acc_x.at[idxd[p]],
                                  ssem[p]).wait()

        live = [False] * NBUF
        for blk in range(N_BLK):
            for p in range(NBUF):
                if live[p]:
                    drain(p)
                    live[p] = False
            pltpu.sync_copy(src_h.at[wid, blk], ibs)
            pltpu.sync_copy(dst_h.at[wid, blk], ibd)
            for j in range(CPB):
                p = j % NBUF
                if live[p]:
                    drain(p)
                pltpu.async_copy(x_h.at[ibs.at[j]], rows[p], gsem[p])
                for g in range(K // LANES):
                    idxd[p][pl.ds(g * LANES, LANES)] = (
                        ibd[j, pl.ds(g * LANES, LANES)])
                pltpu.make_async_copy(x_h.at[ibs.at[j]], rows[p],
                                      gsem[p]).wait()
                pltpu.sync_copy(rows[p], acc_x.at[idxd[p]], add=True)
        for p in range(NBUF):
            if live[p]:
                drain(p)
        plsc.subcore_barrier()

        pltpu.sync_copy(acc_x.at[pl.ds(r0, ROWS_PER_TILE)],
                        part_x_h.at[pl.ds(out0, ROWS_PER_TILE)])

    call = pl.kernel(
        body,
        out_type=jax.ShapeDtypeStruct((NC * N_PAD, D), jnp.float32),
        mesh=mesh,
        scratch_types=(
            [pltpu.VMEM_SHARED((N_PAD, D), jnp.float32)]
            + [pltpu.VMEM((CPB, K), jnp.int32) for _ in range(2)]
            + [pltpu.VMEM((K, D), jnp.float32) for _ in range(NBUF)]
            + [pltpu.VMEM((K,), jnp.int32) for _ in range(NBUF)]
            + [pltpu.SemaphoreType.DMA for _ in range(2 * NBUF)]
        ),
    )
    return call(x, src4, dst4)


def _sc_count(dst4):
    mesh = plsc.VectorSubcoreMesh(core_axis_name="c", subcore_axis_name="s")

    def body(dst_h, part_c_h, acc_c, ibd, rows, ones_v, id0, id1,
             ssem0, ssem1):
        idxd = (id0, id1)
        ssem = (ssem0, ssem1)
        c = lax.axis_index("c")
        s = lax.axis_index("s")
        wid = c * NS + s
        r0 = s * ROWS_PER_TILE
        out0 = c * N_PAD + r0

        zv = jnp.zeros((LANES,), jnp.float32)
        ov = jnp.ones((LANES,), jnp.float32)

        def zrow(i, carry):
            for l in range(D // LANES):
                rows[i, pl.ds(l * LANES, LANES)] = zv
                ones_v[i, pl.ds(l * LANES, LANES)] = ov
            return carry

        lax.fori_loop(0, K, zrow, 0)

        for q in range(ROWS_PER_TILE // K):
            pltpu.sync_copy(rows, acc_c.at[pl.ds(r0 + q * K, K)])
        plsc.subcore_barrier()

        def drain(p):
            pltpu.make_async_copy(ones_v, acc_c.at[idxd[p]], ssem[p]).wait()

        live = [False, False]
        for blk in range(N_BLK):
            pltpu.sync_copy(dst_h.at[wid, blk], ibd)
            for j in range(CPB):
                p = j % 2
                if live[p]:
                    drain(p)
                for g in range(K // LANES):
                    idxd[p][pl.ds(g * LANES, LANES)] = (
                        ibd[j, pl.ds(g * LANES, LANES)])
                pltpu.sync_copy(ones_v, acc_c.at[idxd[p]], add=True)
            for p in range(2):
                if live[p]:
                    drain(p)
                    live[p] = False
        plsc.subcore_barrier()

        pltpu.sync_copy(acc_c.at[pl.ds(r0, ROWS_PER_TILE)],
                        part_c_h.at[pl.ds(out0, ROWS_PER_TILE)])

    call = pl.kernel(
        body,
        out_type=jax.ShapeDtypeStruct((NC * N_PAD, D), jnp.float32),
        mesh=mesh,
        scratch_types=[
            pltpu.VMEM_SHARED((N_PAD, D), jnp.float32),
            pltpu.VMEM((CPB, K), jnp.int32),
            pltpu.VMEM((K, D), jnp.float32),
            pltpu.VMEM((K, D), jnp.float32),
            pltpu.VMEM((K,), jnp.int32),
            pltpu.VMEM((K,), jnp.int32),
            pltpu.SemaphoreType.DMA,
            pltpu.SemaphoreType.DMA,
        ],
    )
    return call(dst4)


def _tc_tail(px0, px1, pc0, pc1, x, WlT, bl, WrT, WlnT):
    def body(px0_ref, px1_ref, pc0_ref, pc1_ref, x_ref,
             wl_ref, bl_ref, wr_ref, wln_ref, o_ref):
        agg = px0_ref[...] + px1_ref[...]
        cnt = pc0_ref[...] + pc1_ref[...]
        mean = agg / jnp.maximum(cnt, 1.0)
        xb = x_ref[...]
        h = jnp.dot(mean, wl_ref[...], preferred_element_type=jnp.float32)
        h = h + bl_ref[...] + jnp.dot(xb, wr_ref[...],
                                      preferred_element_type=jnp.float32)
        h = jnp.maximum(h, 0.0)
        o = jnp.dot(h, wln_ref[...], preferred_element_type=jnp.float32) + xb
        o_ref[...] = jnp.maximum(o, 0.0)

    grid = (N_NODES // TC_BLK,)
    return pl.pallas_call(
        body,
        grid=grid,
        in_specs=[
            pl.BlockSpec((TC_BLK, D), lambda i: (i, 0)),
            pl.BlockSpec((TC_BLK, D), lambda i: (i, 0)),
            pl.BlockSpec((TC_BLK, D), lambda i: (i, 0)),
            pl.BlockSpec((TC_BLK, D), lambda i: (i, 0)),
            pl.BlockSpec((TC_BLK, D), lambda i: (i, 0)),
            pl.BlockSpec((D, D), lambda i: (0, 0)),
            pl.BlockSpec((1, D), lambda i: (0, 0)),
            pl.BlockSpec((D, D), lambda i: (0, 0)),
            pl.BlockSpec((D, D), lambda i: (0, 0)),
        ],
        out_specs=pl.BlockSpec((TC_BLK, D), lambda i: (i, 0)),
        out_shape=jax.ShapeDtypeStruct((N_NODES, D), jnp.float32),
    )(px0, px1, pc0, pc1, x, WlT, bl, WrT, WlnT)


TC_BLK = 1000


def kernel(x, edge_index, W_l, b_l, W_r, W_ln):
    src = edge_index[0].astype(jnp.int32)
    dst = edge_index[1].astype(jnp.int32)
    src4 = src.reshape(NC * NS, N_BLK, CPB, K)
    dst4 = dst.reshape(NC * NS, N_BLK, CPB, K)
    part_x = _sc_aggregate(x, src4, dst4)
    part_c = _sc_count(dst4)
    px0, px1 = part_x[:N_PAD], part_x[N_PAD:]
    pc0, pc1 = part_c[:N_PAD], part_c[N_PAD:]
    return _tc_tail(px0, px1, pc0, pc1, x,
                    W_l.T, b_l.reshape(1, D), W_r.T, W_ln.T)


# speedup vs baseline: 1.4700x; 1.1646x over previous
import jax
import jax.numpy as jnp
from jax import lax
from jax.experimental import pallas as pl
from jax.experimental.pallas import tpu as pltpu
from jax.experimental.pallas import tpu_sc as plsc

N_NODES = 10000
N_EDGES = 320000
D = 128

NC = 2
NS = 16
LANES = 16
K = 80
E_PER_CORE = N_EDGES // NC
E_PER_TILE = E_PER_CORE // NS
N_CHUNKS = E_PER_TILE // K
E_PT_PAD = E_PER_TILE
NBUF = 2
N_BLK = 5
CPB = N_CHUNKS // N_BLK
N_PAD = 10240
ROWS_PER_TILE = N_PAD // NS
CW = 16


def _sc_aggregate(x, src4, dst4):
    mesh = plsc.VectorSubcoreMesh(core_axis_name="c", subcore_axis_name="s")

    def body(x_h, src_h, dst_h, part_x_h, acc_x,
             ibs, ibd, rows0, rows1, id0, id1, gsem0, gsem1, ssem0, ssem1):
        rows = (rows0, rows1)
        idxd = (id0, id1)
        gsem = (gsem0, gsem1)
        ssem = (ssem0, ssem1)

        c = lax.axis_index("c")
        s = lax.axis_index("s")
        wid = c * NS + s
        r0 = s * ROWS_PER_TILE
        out0 = c * N_PAD + r0

        zv = jnp.zeros((LANES,), jnp.float32)

        def zrow(i, carry):
            for l in range(D // LANES):
                rows0[i, pl.ds(l * LANES, LANES)] = zv
            return carry

        lax.fori_loop(0, K, zrow, 0)

        for q in range(ROWS_PER_TILE // K):
            pltpu.sync_copy(rows0, acc_x.at[pl.ds(r0 + q * K, K)])
        plsc.subcore_barrier()

        def drain(p):
            pltpu.make_async_copy(rows[p], acc_x.at[idxd[p]],
                                  ssem[p]).wait()

        live = [False] * NBUF
        for blk in range(N_BLK):
            for p in range(NBUF):
                if live[p]:
                    drain(p)
                    live[p] = False
            pltpu.sync_copy(src_h.at[wid, blk], ibs)
            pltpu.sync_copy(dst_h.at[wid, blk], ibd)
            for j in range(CPB):
                p = j % NBUF
                if live[p]:
                    drain(p)
                pltpu.async_copy(x_h.at[ibs.at[j]], rows[p], gsem[p])
                for g in range(K // LANES):
                    idxd[p][pl.ds(g * LANES, LANES)] = (
                        ibd[j, pl.ds(g * LANES, LANES)])
                pltpu.make_async_copy(x_h.at[ibs.at[j]], rows[p],
                                      gsem[p]).wait()
                pltpu.async_copy(rows[p], acc_x.at[idxd[p]], ssem[p],
                                 add=True)
                live[p] = True
        for p in range(NBUF):
            if live[p]:
                drain(p)
        plsc.subcore_barrier()

        pltpu.sync_copy(acc_x.at[pl.ds(r0, ROWS_PER_TILE)],
                        part_x_h.at[pl.ds(out0, ROWS_PER_TILE)])

    call = pl.kernel(
        body,
        out_type=jax.ShapeDtypeStruct((NC * N_PAD, D), jnp.float32),
        mesh=mesh,
        scratch_types=(
            [pltpu.VMEM_SHARED((N_PAD, D), jnp.float32)]
            + [pltpu.VMEM((CPB, K), jnp.int32) for _ in range(2)]
            + [pltpu.VMEM((K, D), jnp.float32) for _ in range(NBUF)]
            + [pltpu.VMEM((K,), jnp.int32) for _ in range(NBUF)]
            + [pltpu.SemaphoreType.DMA for _ in range(2 * NBUF)]
        ),
    )
    return call(x, src4, dst4)


def _sc_count(dst4):
    mesh = plsc.VectorSubcoreMesh(core_axis_name="c", subcore_axis_name="s")

    def body(dst_h, part_c_h, acc_c, ibd, rows, ones_v, id0, id1,
             ssem0, ssem1):
        idxd = (id0, id1)
        ssem = (ssem0, ssem1)
        c = lax.axis_index("c")
        s = lax.axis_index("s")
        wid = c * NS + s
        r0 = s * ROWS_PER_TILE
        out0 = c * N_PAD + r0

        zv = jnp.zeros((LANES,), jnp.float32)
        ov = jnp.ones((LANES,), jnp.float32)

        def zrow(i, carry):
            for l in range(D // LANES):
                rows[i, pl.ds(l * LANES, LANES)] = zv
                ones_v[i, pl.ds(l * LANES, LANES)] = ov
            return carry

        lax.fori_loop(0, K, zrow, 0)

        for q in range(ROWS_PER_TILE // K):
            pltpu.sync_copy(rows, acc_c.at[pl.ds(r0 + q * K, K)])
        plsc.subcore_barrier()

        def drain(p):
            pltpu.make_async_copy(ones_v, acc_c.at[idxd[p]], ssem[p]).wait()

        live = [False, False]
        for blk in range(N_BLK):
            pltpu.sync_copy(dst_h.at[wid, blk], ibd)
            for j in range(CPB):
                p = j % 2
                if live[p]:
                    drain(p)
                for g in range(K // LANES):
                    idxd[p][pl.ds(g * LANES, LANES)] = (
                        ibd[j, pl.ds(g * LANES, LANES)])
                pltpu.async_copy(ones_v, acc_c.at[idxd[p]], ssem[p],
                                 add=True)
                live[p] = True
            for p in range(2):
                if live[p]:
                    drain(p)
                    live[p] = False
        plsc.subcore_barrier()

        pltpu.sync_copy(acc_c.at[pl.ds(r0, ROWS_PER_TILE)],
                        part_c_h.at[pl.ds(out0, ROWS_PER_TILE)])

    call = pl.kernel(
        body,
        out_type=jax.ShapeDtypeStruct((NC * N_PAD, D), jnp.float32),
        mesh=mesh,
        scratch_types=[
            pltpu.VMEM_SHARED((N_PAD, D), jnp.float32),
            pltpu.VMEM((CPB, K), jnp.int32),
            pltpu.VMEM((K, D), jnp.float32),
            pltpu.VMEM((K, D), jnp.float32),
            pltpu.VMEM((K,), jnp.int32),
            pltpu.VMEM((K,), jnp.int32),
            pltpu.SemaphoreType.DMA,
            pltpu.SemaphoreType.DMA,
        ],
    )
    return call(dst4)


def _tc_tail(px0, px1, pc0, pc1, x, WlT, bl, WrT, WlnT):
    def body(px0_ref, px1_ref, pc0_ref, pc1_ref, x_ref,
             wl_ref, bl_ref, wr_ref, wln_ref, o_ref):
        agg = px0_ref[...] + px1_ref[...]
        cnt = pc0_ref[...] + pc1_ref[...]
        mean = agg / jnp.maximum(cnt, 1.0)
        xb = x_ref[...]
        h = jnp.dot(mean, wl_ref[...], preferred_element_type=jnp.float32)
        h = h + bl_ref[...] + jnp.dot(xb, wr_ref[...],
                                      preferred_element_type=jnp.float32)
        h = jnp.maximum(h, 0.0)
        o = jnp.dot(h, wln_ref[...], preferred_element_type=jnp.float32) + xb
        o_ref[...] = jnp.maximum(o, 0.0)

    grid = (N_NODES // TC_BLK,)
    return pl.pallas_call(
        body,
        grid=grid,
        in_specs=[
            pl.BlockSpec((TC_BLK, D), lambda i: (i, 0)),
            pl.BlockSpec((TC_BLK, D), lambda i: (i, 0)),
            pl.BlockSpec((TC_BLK, D), lambda i: (i, 0)),
            pl.BlockSpec((TC_BLK, D), lambda i: (i, 0)),
            pl.BlockSpec((TC_BLK, D), lambda i: (i, 0)),
            pl.BlockSpec((D, D), lambda i: (0, 0)),
            pl.BlockSpec((1, D), lambda i: (0, 0)),
            pl.BlockSpec((D, D), lambda i: (0, 0)),
            pl.BlockSpec((D, D), lambda i: (0, 0)),
        ],
        out_specs=pl.BlockSpec((TC_BLK, D), lambda i: (i, 0)),
        out_shape=jax.ShapeDtypeStruct((N_NODES, D), jnp.float32),
    )(px0, px1, pc0, pc1, x, WlT, bl, WrT, WlnT)


TC_BLK = 1000


def kernel(x, edge_index, W_l, b_l, W_r, W_ln):
    src = edge_index[0].astype(jnp.int32)
    dst = edge_index[1].astype(jnp.int32)
    src4 = src.reshape(NC * NS, N_BLK, CPB, K)
    dst4 = dst.reshape(NC * NS, N_BLK, CPB, K)
    part_x = _sc_aggregate(x, src4, dst4)
    part_c = _sc_count(dst4)
    px0, px1 = part_x[:N_PAD], part_x[N_PAD:]
    pc0, pc1 = part_c[:N_PAD], part_c[N_PAD:]
    return _tc_tail(px0, px1, pc0, pc1, x,
                    W_l.T, b_l.reshape(1, D), W_r.T, W_ln.T)


# speedup vs baseline: 1.4780x; 1.0054x over previous
import jax
import jax.numpy as jnp
from jax import lax
from jax.experimental import pallas as pl
from jax.experimental.pallas import tpu as pltpu
from jax.experimental.pallas import tpu_sc as plsc

N_NODES = 10000
N_EDGES = 320000
D = 128

NC = 2
NS = 16
LANES = 16
K = 80
E_PER_CORE = N_EDGES // NC
E_PER_TILE = E_PER_CORE // NS
N_CHUNKS = E_PER_TILE // K
E_PT_PAD = E_PER_TILE
NBUF = 3
NBUF_C = 4
N_BLK = 5
CPB = N_CHUNKS // N_BLK
N_PAD = 10240
ROWS_PER_TILE = N_PAD // NS
CW = 16


def _sc_aggregate(x, src4, dst4):
    mesh = plsc.VectorSubcoreMesh(core_axis_name="c", subcore_axis_name="s")

    def body(x_h, src_h, dst_h, part_x_h, acc_x, ibs, ibd, *bufs):
        rows = bufs[0:NBUF]
        idxd = bufs[NBUF:2 * NBUF]
        gsem = bufs[2 * NBUF:3 * NBUF]
        ssem = bufs[3 * NBUF:4 * NBUF]

        c = lax.axis_index("c")
        s = lax.axis_index("s")
        wid = c * NS + s
        r0 = s * ROWS_PER_TILE
        out0 = c * N_PAD + r0

        zv = jnp.zeros((LANES,), jnp.float32)

        def zrow(i, carry):
            for l in range(D // LANES):
                rows[0][i, pl.ds(l * LANES, LANES)] = zv
            return carry

        lax.fori_loop(0, K, zrow, 0)

        for q in range(ROWS_PER_TILE // K):
            pltpu.sync_copy(rows[0], acc_x.at[pl.ds(r0 + q * K, K)])
        plsc.subcore_barrier()

        def drain(p):
            pltpu.make_async_copy(rows[p], acc_x.at[idxd[p]],
                                  ssem[p]).wait()

        live = [False] * NBUF
        for blk in range(N_BLK):
            for p in range(NBUF):
                if live[p]:
                    drain(p)
                    live[p] = False
            pltpu.sync_copy(src_h.at[wid, blk], ibs)
            pltpu.sync_copy(dst_h.at[wid, blk], ibd)
            for j in range(CPB):
                p = j % NBUF
                if live[p]:
                    drain(p)
                pltpu.async_copy(x_h.at[ibs.at[j]], rows[p], gsem[p])
                for g in range(K // LANES):
                    idxd[p][pl.ds(g * LANES, LANES)] = (
                        ibd[j, pl.ds(g * LANES, LANES)])
                pltpu.make_async_copy(x_h.at[ibs.at[j]], rows[p],
                                      gsem[p]).wait()
                pltpu.async_copy(rows[p], acc_x.at[idxd[p]], ssem[p],
                                 add=True)
                live[p] = True
        for p in range(NBUF):
            if live[p]:
                drain(p)
        plsc.subcore_barrier()

        pltpu.sync_copy(acc_x.at[pl.ds(r0, ROWS_PER_TILE)],
                        part_x_h.at[pl.ds(out0, ROWS_PER_TILE)])

    call = pl.kernel(
        body,
        out_type=jax.ShapeDtypeStruct((NC * N_PAD, D), jnp.float32),
        mesh=mesh,
        scratch_types=(
            [pltpu.VMEM_SHARED((N_PAD, D), jnp.float32)]
            + [pltpu.VMEM((CPB, K), jnp.int32) for _ in range(2)]
            + [pltpu.VMEM((K, D), jnp.float32) for _ in range(NBUF)]
            + [pltpu.VMEM((K,), jnp.int32) for _ in range(NBUF)]
            + [pltpu.SemaphoreType.DMA for _ in range(NBUF)]
            + [pltpu.SemaphoreType.DMA for _ in range(NBUF)]
        ),
    )
    return call(x, src4, dst4)


def _sc_count(dst4):
    mesh = plsc.VectorSubcoreMesh(core_axis_name="c", subcore_axis_name="s")

    def body(dst_h, part_c_h, acc_c, ibd, rows, ones_v, *bufs):
        idxd = bufs[0:NBUF_C]
        ssem = bufs[NBUF_C:2 * NBUF_C]
        c = lax.axis_index("c")
        s = lax.axis_index("s")
        wid = c * NS + s
        r0 = s * ROWS_PER_TILE
        out0 = c * N_PAD + r0

        zv = jnp.zeros((LANES,), jnp.float32)
        ov = jnp.ones((LANES,), jnp.float32)

        def zrow(i, carry):
            for l in range(D // LANES):
                rows[i, pl.ds(l * LANES, LANES)] = zv
                ones_v[i, pl.ds(l * LANES, LANES)] = ov
            return carry

        lax.fori_loop(0, K, zrow, 0)

        for q in range(ROWS_PER_TILE // K):
            pltpu.sync_copy(rows, acc_c.at[pl.ds(r0 + q * K, K)])
        plsc.subcore_barrier()

        def drain(p):
            pltpu.make_async_copy(ones_v, acc_c.at[idxd[p]], ssem[p]).wait()

        live = [False] * NBUF_C
        for blk in range(N_BLK):
            pltpu.sync_copy(dst_h.at[wid, blk], ibd)
            for j in range(CPB):
                p = j % NBUF_C
                if live[p]:
                    drain(p)
                for g in range(K // LANES):
                    idxd[p][pl.ds(g * LANES, LANES)] = (
                        ibd[j, pl.ds(g * LANES, LANES)])
                pltpu.async_copy(ones_v, acc_c.at[idxd[p]], ssem[p],
                                 add=True)
                live[p] = True
            for p in range(NBUF_C):
                if live[p]:
                    drain(p)
                    live[p] = False
        plsc.subcore_barrier()

        pltpu.sync_copy(acc_c.at[pl.ds(r0, ROWS_PER_TILE)],
                        part_c_h.at[pl.ds(out0, ROWS_PER_TILE)])

    call = pl.kernel(
        body,
        out_type=jax.ShapeDtypeStruct((NC * N_PAD, D), jnp.float32),
        mesh=mesh,
        scratch_types=[
            pltpu.VMEM_SHARED((N_PAD, D), jnp.float32),
            pltpu.VMEM((CPB, K), jnp.int32),
            pltpu.VMEM((K, D), jnp.float32),
            pltpu.VMEM((K, D), jnp.float32),
        ] + [pltpu.VMEM((K,), jnp.int32) for _ in range(NBUF_C)]
          + [pltpu.SemaphoreType.DMA for _ in range(NBUF_C)],
    )
    return call(dst4)


def _tc_tail(px0, px1, pc0, pc1, x, WlT, bl, WrT, WlnT):
    def body(px0_ref, px1_ref, pc0_ref, pc1_ref, x_ref,
             wl_ref, bl_ref, wr_ref, wln_ref, o_ref):
        agg = px0_ref[...] + px1_ref[...]
        cnt = pc0_ref[...] + pc1_ref[...]
        mean = agg / jnp.maximum(cnt, 1.0)
        xb = x_ref[...]
        h = jnp.dot(mean, wl_ref[...], preferred_element_type=jnp.float32)
        h = h + bl_ref[...] + jnp.dot(xb, wr_ref[...],
                                      preferred_element_type=jnp.float32)
        h = jnp.maximum(h, 0.0)
        o = jnp.dot(h, wln_ref[...], preferred_element_type=jnp.float32) + xb
        o_ref[...] = jnp.maximum(o, 0.0)

    grid = (N_NODES // TC_BLK,)
    return pl.pallas_call(
        body,
        grid=grid,
        in_specs=[
            pl.BlockSpec((TC_BLK, D), lambda i: (i, 0)),
            pl.BlockSpec((TC_BLK, D), lambda i: (i, 0)),
            pl.BlockSpec((TC_BLK, D), lambda i: (i, 0)),
            pl.BlockSpec((TC_BLK, D), lambda i: (i, 0)),
            pl.BlockSpec((TC_BLK, D), lambda i: (i, 0)),
            pl.BlockSpec((D, D), lambda i: (0, 0)),
            pl.BlockSpec((1, D), lambda i: (0, 0)),
            pl.BlockSpec((D, D), lambda i: (0, 0)),
            pl.BlockSpec((D, D), lambda i: (0, 0)),
        ],
        out_specs=pl.BlockSpec((TC_BLK, D), lambda i: (i, 0)),
        out_shape=jax.ShapeDtypeStruct((N_NODES, D), jnp.float32),
    )(px0, px1, pc0, pc1, x, WlT, bl, WrT, WlnT)


TC_BLK = 1000


def kernel(x, edge_index, W_l, b_l, W_r, W_ln):
    src = edge_index[0].astype(jnp.int32)
    dst = edge_index[1].astype(jnp.int32)
    src4 = src.reshape(NC * NS, N_BLK, CPB, K)
    dst4 = dst.reshape(NC * NS, N_BLK, CPB, K)
    part_x = _sc_aggregate(x, src4, dst4)
    part_c = _sc_count(dst4)
    px0, px1 = part_x[:N_PAD], part_x[N_PAD:]
    pc0, pc1 = part_c[:N_PAD], part_c[N_PAD:]
    return _tc_tail(px0, px1, pc0, pc1, x,
                    W_l.T, b_l.reshape(1, D), W_r.T, W_ln.T)
